# baseline (speedup 1.0000x reference)
import functools

import jax
import jax.numpy as jnp
from jax import lax
from jax.experimental import pallas as pl
from jax.experimental.pallas import tpu as pltpu
from jax.experimental.pallas import tpu_sc as plsc


def kernel(inputs, word_table, pos_table):
    B, S = inputs.shape
    V, D = word_table.shape
    info = plsc.get_sparse_core_info()
    NC, NS, L = info.num_cores, info.num_subcores, info.num_lanes
    NW = NC * NS
    assert B % NW == 0 and D % L == 0 and S % 8 == 0 and V % 2 == 0
    seqs_per_w = B // NW

    TB = 128
    n_full = V // TB
    tail = V - n_full * TB
    blocks_per_w = (n_full + NW - 1) // NW
    n_pairs = (blocks_per_w + 2) // 2
    NCH = 2 * D // L

    mesh = plsc.VectorSubcoreMesh(core_axis_name="c", subcore_axis_name="s")
    tableT = word_table.T
    tail2 = word_table[n_full * TB:].reshape(tail // 2, 2 * D) if tail else None

    @functools.partial(
        pl.kernel,
        out_type=jax.ShapeDtypeStruct((V // 2, 2 * D), jnp.float32),
        mesh=mesh,
        scratch_types=[
            pltpu.VMEM((2, D, TB), jnp.float32),
            pltpu.VMEM((D, 2 * D), jnp.float32),
            pltpu.SemaphoreType.DMA,
        ],
        compiler_params=pltpu.CompilerParams(
            use_tc_tiling_on_sc=True, needs_layout_passes=False),
    )
    def ktrans(tT, tail2_hbm, t128, tin, tout, sem):
        wid = lax.axis_index("s") * NC + lax.axis_index("c")
        iv = lax.iota(jnp.int32, L)

        def issue(blk, slot):
            v0 = blk * TB
            for d8 in range(D // 8):
                pltpu.async_copy(
                    tT.at[pl.ds(d8 * 8, 8), pl.ds(v0, TB)],
                    tin.at[slot, pl.ds(d8 * 8, 8), :], sem)

        def drain(slot):
            for d8 in range(D // 8):
                pltpu.make_async_copy(
                    tT.at[pl.ds(0, 8), pl.ds(0, TB)],
                    tin.at[slot, pl.ds(d8 * 8, 8), :], sem).wait()

        def transpose_rows(slot, r):
            for c16 in range(NCH):
                d0 = (c16 * L) % D
                koff = (c16 * L) // D
                kvec = jnp.broadcast_to(2 * r + koff, (L,))
                g = plsc.load_gather(tin.at[slot], [d0 + iv, kvec])
                tout[r, pl.ds(c16 * L, L)] = g

        first = wid * blocks_per_w
        issue(first, 0)

        def pair_body(g, c):
            for b in range(2):
                i = g * 2 + b
                blk = first + i
                valid = jnp.logical_and(i < blocks_per_w, blk < n_full)
                nblk = blk + 1
                nvalid = jnp.logical_and(i + 1 < blocks_per_w, nblk < n_full)

                @pl.when(valid)
                def _():
                    drain(b)

                @pl.when(nvalid)
                def _():
                    issue(nblk, 1 - b)

                @pl.when(valid)
                def _():
                    def trow(r, c2):
                        transpose_rows(b, r)
                        return c2

                    lax.fori_loop(0, D, trow, 0)
                    pltpu.sync_copy(tout, t128.at[pl.ds(blk * D, D), :])
            return c

        lax.fori_loop(0, n_pairs, pair_body, 0)

        if tail:
            @pl.when(wid == 0)
            def _():
                pltpu.sync_copy(tail2_hbm, tout.at[pl.ds(0, tail // 2), :])
                pltpu.sync_copy(tout.at[pl.ds(0, tail // 2), :],
                                t128.at[pl.ds(n_full * D, tail // 2), :])

    t128 = ktrans(tableT, tail2)
    t_lin = t128.reshape(V, D)

    idx_flat = inputs.reshape(B * S)

    @functools.partial(
        pl.kernel,
        out_type=jax.ShapeDtypeStruct((B * S, D), jnp.float32),
        mesh=mesh,
        scratch_types=[
            pltpu.VMEM((S,), jnp.int32),
            pltpu.VMEM((S, D), jnp.float32),
            pltpu.VMEM((S, D), jnp.float32),
            pltpu.SemaphoreType.DMA,
        ],
        compiler_params=pltpu.CompilerParams(use_tc_tiling_on_sc=False),
    )
    def kgather(idx_hbm, table_hbm, pos_hbm, out_hbm, idx_v, rows_v, pos_v,
                gsem):
        wid = lax.axis_index("s") * NC + lax.axis_index("c")
        base = wid * seqs_per_w * S
        pltpu.sync_copy(pos_hbm, pos_v)

        def body(b, carry):
            start = base + b * S
            pltpu.sync_copy(idx_hbm.at[pl.ds(start, S)], idx_v)
            pltpu.async_copy(table_hbm.at[idx_v], rows_v, gsem).wait()

            def add_row(srow, c2):
                for j in range(D // L):
                    sl = pl.ds(j * L, L)
                    rows_v[srow, sl] = rows_v[srow, sl] + pos_v[srow, sl]
                return c2

            lax.fori_loop(0, S, add_row, 0)
            pltpu.sync_copy(rows_v, out_hbm.at[pl.ds(start, S)])
            return carry

        lax.fori_loop(0, seqs_per_w, body, 0)

    out = kgather(idx_flat, t_lin, pos_table)
    return out.reshape(B, S, D)


# speedup vs baseline: 2.2904x; 2.2904x over previous
import functools

import jax
import jax.numpy as jnp
from jax import lax
from jax.experimental import pallas as pl
from jax.experimental.pallas import tpu as pltpu
from jax.experimental.pallas import tpu_sc as plsc


def kernel(inputs, word_table, pos_table):
    B, S = inputs.shape
    V, D = word_table.shape
    info = plsc.get_sparse_core_info()
    NC, NS, L = info.num_cores, info.num_subcores, info.num_lanes
    NW = NC * NS
    assert B % NW == 0 and D % L == 0 and (S * D) % 8 == 0
    seqs_per_w = B // NW
    assert seqs_per_w % 2 == 0

    idx_flat = inputs.reshape(B * S)
    mesh = plsc.VectorSubcoreMesh(core_axis_name="c", subcore_axis_name="s")

    @functools.partial(
        pl.kernel,
        out_type=jax.ShapeDtypeStruct((B * S, D), jnp.float32),
        mesh=mesh,
        scratch_types=[
            pltpu.VMEM((2, S), jnp.int32),
            pltpu.VMEM((2, S, D), jnp.float32),
            pltpu.VMEM((S, D), jnp.float32),
            pltpu.SemaphoreType.DMA,
            pltpu.SemaphoreType.DMA,
        ],
        compiler_params=pltpu.CompilerParams(use_tc_tiling_on_sc=False),
    )
    def emb_kernel(idx_hbm, table_hbm, pos_hbm, out_hbm,
                   idx_v, rows_v, pos_v, gsem, osem):
        wid = lax.axis_index("s") * NC + lax.axis_index("c")
        base = wid * seqs_per_w * S
        pltpu.sync_copy(pos_hbm, pos_v)

        def start_gather(b, slot):
            start = base + b * S
            pltpu.sync_copy(idx_hbm.at[pl.ds(start, S)], idx_v.at[slot])
            pltpu.async_copy(table_hbm.at[idx_v.at[slot]],
                             rows_v.at[slot], gsem)

        def gather_wait(slot):
            pltpu.make_async_copy(table_hbm.at[idx_v.at[slot]],
                                  rows_v.at[slot], gsem).wait()

        def out_wait(slot):
            pltpu.make_async_copy(rows_v.at[slot],
                                  out_hbm.at[pl.ds(0, S)], osem).wait()

        start_gather(0, 0)

        def pair_body(g, carry):
            for sl in range(2):
                b = g * 2 + sl
                gather_wait(sl)

                @pl.when(b + 1 < seqs_per_w)
                def _():
                    start_gather(b + 1, 1 - sl)

                def add_row(srow, c2):
                    for j in range(D // L):
                        cs = pl.ds(j * L, L)
                        rows_v[sl, srow, cs] = (
                            rows_v[sl, srow, cs] + pos_v[srow, cs])
                    return c2

                lax.fori_loop(0, S, add_row, 0)

                @pl.when(b >= 2)
                def _():
                    out_wait(sl)

                start = base + b * S
                pltpu.async_copy(rows_v.at[sl],
                                 out_hbm.at[pl.ds(start, S)], osem)
            return carry

        lax.fori_loop(0, seqs_per_w // 2, pair_body, 0)
        out_wait(0)
        out_wait(1)

    out = emb_kernel(idx_flat, word_table, pos_table)
    return out.reshape(B, S, D)
